# baseline (speedup 1.0000x reference)
import functools

import jax
import jax.numpy as jnp
from jax import lax
from jax.experimental import pallas as pl
from jax.experimental.pallas import tpu as pltpu
from jax.experimental.pallas import tpu_sc as plsc

_N = 4096
_E = 65536
_D = 16
_K_ITERS = 16
_BM = 512
_NW = 32
_CHUNK = 128
_NCHUNK = _E // (_NW * _CHUNK)
_ROWS_PER_TILE = _N // 16


def _solver_body(alpha_ref, L_ref, V_ref, out_ref, y0, y1):
    k = pl.program_id(0)
    i = pl.program_id(1)
    alpha = alpha_ref[0, 0]

    @pl.when(jnp.logical_and(k == 0, i == 0))
    def _():
        y0[...] = V_ref[...]

    def step(src, dst):
        blk = V_ref[pl.ds(i * _BM, _BM), :] + alpha * jnp.dot(
            L_ref[...], src[...], preferred_element_type=jnp.float32,
            precision=lax.Precision.HIGHEST)
        dst[pl.ds(i * _BM, _BM), :] = blk
        out_ref[...] = (1.0 - alpha) * blk

    @pl.when(k % 2 == 0)
    def _():
        step(y0, y1)

    @pl.when(k % 2 == 1)
    def _():
        step(y1, y0)


def _neumann_solve(alpha, laplacian, V):
    return pl.pallas_call(
        _solver_body,
        grid=(_K_ITERS, _N // _BM),
        in_specs=[
            pl.BlockSpec(memory_space=pltpu.SMEM),
            pl.BlockSpec((_BM, _N), lambda k, i: (i, 0)),
            pl.BlockSpec((_N, 8), lambda k, i: (0, 0)),
        ],
        out_specs=pl.BlockSpec((_BM, 8), lambda k, i: (i, 0)),
        out_shape=jax.ShapeDtypeStruct((_N, 8), jnp.float32),
        scratch_shapes=[
            pltpu.VMEM((_N, 8), jnp.float32),
            pltpu.VMEM((_N, 8), jnp.float32),
        ],
    )(jnp.reshape(alpha, (1, 1)), laplacian, V)


def _sc_scatter_body(src_hbm, dst_hbm, table_hbm, zeros_hbm, out_hbm,
                     srcv, dstv, rows, zrows, acc, sem):
    c = lax.axis_index("c")
    s = lax.axis_index("s")
    wid = s * 2 + c

    pltpu.sync_copy(zeros_hbm, zrows)
    pltpu.sync_copy(zrows, acc.at[pl.ds(s * _ROWS_PER_TILE, _ROWS_PER_TILE)])
    plsc.subcore_barrier()

    pltpu.sync_copy(src_hbm.at[wid], srcv)
    pltpu.sync_copy(dst_hbm.at[wid], dstv)

    def chunk(j, carry):
        pltpu.async_copy(table_hbm.at[srcv.at[j]], rows, sem).wait()
        pltpu.sync_copy(rows, acc.at[dstv.at[j]], add=True)
        return carry

    lax.fori_loop(0, _NCHUNK, chunk, 0)
    plsc.subcore_barrier()

    base = c * _N + s * _ROWS_PER_TILE
    pltpu.sync_copy(acc.at[pl.ds(s * _ROWS_PER_TILE, _ROWS_PER_TILE)],
                    out_hbm.at[pl.ds(base, _ROWS_PER_TILE)])


def _sc_scatter(src3, dst3, table, zeros_hbm):
    mesh = plsc.VectorSubcoreMesh(core_axis_name="c", subcore_axis_name="s")
    f = pl.kernel(
        _sc_scatter_body,
        out_type=jax.ShapeDtypeStruct((2 * _N, _D), jnp.float32),
        mesh=mesh,
        scratch_types=[
            pltpu.VMEM((_NCHUNK, _CHUNK), jnp.int32),
            pltpu.VMEM((_NCHUNK, _CHUNK), jnp.int32),
            pltpu.VMEM((_CHUNK, _D), jnp.float32),
            pltpu.VMEM((_ROWS_PER_TILE, _D), jnp.float32),
            pltpu.VMEM_SHARED((_N, _D), jnp.float32),
            pltpu.SemaphoreType.DMA,
        ],
        compiler_params=pltpu.CompilerParams(use_tc_tiling_on_sc=False),
    )
    return f(src3, dst3, table, zeros_hbm)


def _prep_body(degp_ref, Y_ref, dv_ref, table1_ref):
    deg = degp_ref[0:_N, 0:1] + degp_ref[_N:2 * _N, 0:1] + 1.0
    dinv = lax.rsqrt(deg)
    table1_ref[...] = jnp.concatenate(
        [dinv * dv_ref[...], dinv * Y_ref[:, 0:3], dinv,
         jnp.zeros((_N, 11), jnp.float32)], axis=1)


def _mid_body(praw_ref, table1_ref, W1_ref, b1_ref, W2_ref, Wf_ref,
              table2_ref, s_ref):
    dinv = table1_ref[:, 4:5]
    p_full = dinv * (praw_ref[0:_N, :] + praw_ref[_N:2 * _N, :]) \
        + dinv * table1_ref[...]
    p = p_full[:, 0:4]
    s = p_full[:, 4:5]
    z = jnp.dot(p, W1_ref[...].T, preferred_element_type=jnp.float32,
                precision=lax.Precision.HIGHEST) + s * b1_ref[...]
    h = jnp.maximum(z, 0.0)
    C = jnp.dot(Wf_ref[...], W2_ref[...], preferred_element_type=jnp.float32,
                precision=lax.Precision.HIGHEST)
    g = jnp.dot(h, C.T, preferred_element_type=jnp.float32,
                precision=lax.Precision.HIGHEST)
    table2_ref[...] = jnp.concatenate(
        [dinv * g, jnp.zeros((_N, 14), jnp.float32)], axis=1)
    s_ref[...] = s


def _final_body(qraw_ref, table1_ref, table2_ref, s_ref, Wf_ref, b2_ref,
                bf_ref, out_ref):
    dinv = table1_ref[:, 4:5]
    qsum = qraw_ref[0:_N, 0:2] + qraw_ref[_N:2 * _N, 0:2]
    sb = jnp.dot(b2_ref[...], Wf_ref[...].T,
                 preferred_element_type=jnp.float32)
    out_ref[...] = dinv * qsum + dinv * table2_ref[:, 0:2] \
        + s_ref[...] * sb + bf_ref[...]


def _tc_single(body, out_shapes, *args):
    return pl.pallas_call(
        body,
        out_shape=out_shapes,
    )(*args)


def kernel(alpha, laplacian, num_node, threshold, diff_vec, edge_index,
           W1, b1, W2, b2, Wf, bf):
    n = diff_vec.shape[0]
    v = diff_vec.astype(jnp.float32)
    V3 = jnp.where(v < threshold, threshold, v)
    V4 = jnp.where(v >= threshold, threshold, v)
    V = jnp.concatenate(
        [v[:, None], V3[:, None], V4[:, None], jnp.zeros((n, 5), jnp.float32)],
        axis=1)

    Y = _neumann_solve(alpha, laplacian, V)

    src3 = edge_index[0].reshape(_NW, _NCHUNK, _CHUNK)
    dst3 = edge_index[1].reshape(_NW, _NCHUNK, _CHUNK)
    zeros_hbm = jnp.zeros((_ROWS_PER_TILE, _D), jnp.float32)
    ones_table = jnp.ones((_N, _D), jnp.float32)

    degp = _sc_scatter(src3, src3, ones_table, zeros_hbm)

    table1 = _tc_single(
        _prep_body, jax.ShapeDtypeStruct((_N, _D), jnp.float32),
        degp, Y, v[:, None])

    praw = _sc_scatter(src3, dst3, table1, zeros_hbm)

    table2, s = _tc_single(
        _mid_body,
        [jax.ShapeDtypeStruct((_N, _D), jnp.float32),
         jax.ShapeDtypeStruct((_N, 1), jnp.float32)],
        praw, table1, W1, jnp.reshape(b1, (1, 128)), W2, Wf)

    qraw = _sc_scatter(src3, dst3, table2, zeros_hbm)

    out = _tc_single(
        _final_body, jax.ShapeDtypeStruct((_N, 2), jnp.float32),
        qraw, table1, table2, s, Wf, jnp.reshape(b2, (1, 128)),
        jnp.reshape(bf, (1, 2)))

    return out + (jnp.asarray(num_node) - n).astype(out.dtype)


# speedup vs baseline: 3.4873x; 3.4873x over previous
import functools

import jax
import jax.numpy as jnp
from jax import lax
from jax.experimental import pallas as pl
from jax.experimental.pallas import tpu as pltpu
from jax.experimental.pallas import tpu_sc as plsc

_N = 4096
_E = 65536
_D = 16
_K_ITERS = 16
_BM = 512
_NW = 32
_CHUNK = 128
_NCHUNK = _E // (_NW * _CHUNK)
_ROWS_PER_TILE = _N // 16


_K_RES = 6
_K_REF = 2


def _cast_sweep_body(alpha_ref, L_ref, V_ref, Lhi_ref, Y1_ref):
    alpha = alpha_ref[0, 0]
    Lf = L_ref[...]
    Lhi_ref[...] = Lf.astype(jnp.bfloat16)
    Y1_ref[...] = V_ref[pl.ds(pl.program_id(0) * _BM, _BM), :] + alpha * (
        jnp.dot(Lf, V_ref[...], preferred_element_type=jnp.float32))


def _refine_body(alpha_ref, L_ref, V_ref, Yin_ref, out_ref, y0, y1):
    k = pl.program_id(0)
    i = pl.program_id(1)
    alpha = alpha_ref[0, 0]

    @pl.when(jnp.logical_and(k == 0, i == 0))
    def _():
        y0[...] = Yin_ref[...]

    def step(src, dst):
        Lf = L_ref[...]
        Lhi = Lf.astype(jnp.bfloat16)
        Llo = (Lf - Lhi.astype(jnp.float32)).astype(jnp.bfloat16)
        Ys = src[...]
        Yhi = Ys.astype(jnp.bfloat16)
        Ylo = (Ys - Yhi.astype(jnp.float32)).astype(jnp.bfloat16)
        acc = jnp.dot(Lhi, Yhi, preferred_element_type=jnp.float32)
        acc += jnp.dot(Lhi, Ylo, preferred_element_type=jnp.float32)
        acc += jnp.dot(Llo, Yhi, preferred_element_type=jnp.float32)
        new = V_ref[pl.ds(i * _BM, _BM), :] + alpha * acc
        dst[pl.ds(i * _BM, _BM), :] = new
        out_ref[...] = (1.0 - alpha) * new

    @pl.when(k % 2 == 0)
    def _():
        step(y0, y1)

    @pl.when(k % 2 == 1)
    def _():
        step(y1, y0)


def _neumann_solve(alpha, laplacian, V):
    alpha_s = jnp.reshape(alpha, (1, 1))
    Lhi, Y1 = pl.pallas_call(
        _cast_sweep_body,
        grid=(_N // _BM,),
        in_specs=[
            pl.BlockSpec(memory_space=pltpu.SMEM),
            pl.BlockSpec((_BM, _N), lambda i: (i, 0)),
            pl.BlockSpec((_N, 8), lambda i: (0, 0)),
        ],
        out_specs=[
            pl.BlockSpec((_BM, _N), lambda i: (i, 0)),
            pl.BlockSpec((_BM, 8), lambda i: (i, 0)),
        ],
        out_shape=[
            jax.ShapeDtypeStruct((_N, _N), jnp.bfloat16),
            jax.ShapeDtypeStruct((_N, 8), jnp.float32),
        ],
    )(alpha_s, laplacian, V)

    Y2 = pl.pallas_call(
        _resident_step_wrapper,
        grid=(_K_RES, _N // _BM),
        in_specs=[
            pl.BlockSpec(memory_space=pltpu.SMEM),
            pl.BlockSpec((_N, _N), lambda k, i: (0, 0)),
            pl.BlockSpec((_N, 8), lambda k, i: (0, 0)),
            pl.BlockSpec((_N, 8), lambda k, i: (0, 0)),
        ],
        out_specs=pl.BlockSpec((_BM, 8), lambda k, i: (i, 0)),
        out_shape=jax.ShapeDtypeStruct((_N, 8), jnp.float32),
        scratch_shapes=[
            pltpu.VMEM((_N, 8), jnp.float32),
            pltpu.VMEM((_N, 8), jnp.float32),
        ],
    )(alpha_s, Lhi, V, Y1)

    return pl.pallas_call(
        _refine_body,
        grid=(_K_REF, _N // _BM),
        in_specs=[
            pl.BlockSpec(memory_space=pltpu.SMEM),
            pl.BlockSpec((_BM, _N), lambda k, i: (i, 0)),
            pl.BlockSpec((_N, 8), lambda k, i: (0, 0)),
            pl.BlockSpec((_N, 8), lambda k, i: (0, 0)),
        ],
        out_specs=pl.BlockSpec((_BM, 8), lambda k, i: (i, 0)),
        out_shape=jax.ShapeDtypeStruct((_N, 8), jnp.float32),
        scratch_shapes=[
            pltpu.VMEM((_N, 8), jnp.float32),
            pltpu.VMEM((_N, 8), jnp.float32),
        ],
    )(alpha_s, laplacian, V, Y2)


def _resident_step_wrapper(alpha_ref, Lhi_ref, V_ref, Y1_ref, out_ref, y0, y1):
    k = pl.program_id(0)
    i = pl.program_id(1)
    alpha = alpha_ref[0, 0]

    @pl.when(jnp.logical_and(k == 0, i == 0))
    def _():
        y0[...] = Y1_ref[...]

    def step(src, dst):
        Lblk = Lhi_ref[pl.ds(i * _BM, _BM), :]
        acc = jnp.dot(Lblk, src[...].astype(jnp.bfloat16),
                      preferred_element_type=jnp.float32)
        new = V_ref[pl.ds(i * _BM, _BM), :] + alpha * acc
        dst[pl.ds(i * _BM, _BM), :] = new
        out_ref[...] = new

    @pl.when(k % 2 == 0)
    def _():
        step(y0, y1)

    @pl.when(k % 2 == 1)
    def _():
        step(y1, y0)


def _sc_scatter_body(src_hbm, dst_hbm, table_hbm, zeros_hbm, out_hbm,
                     srcv, dstv, rows, zrows, acc, sem):
    c = lax.axis_index("c")
    s = lax.axis_index("s")
    wid = s * 2 + c

    pltpu.sync_copy(zeros_hbm, zrows)
    pltpu.sync_copy(zrows, acc.at[pl.ds(s * _ROWS_PER_TILE, _ROWS_PER_TILE)])
    plsc.subcore_barrier()

    pltpu.sync_copy(src_hbm.at[wid], srcv)
    pltpu.sync_copy(dst_hbm.at[wid], dstv)

    def chunk(j, carry):
        pltpu.async_copy(table_hbm.at[srcv.at[j]], rows, sem).wait()
        pltpu.sync_copy(rows, acc.at[dstv.at[j]], add=True)
        return carry

    lax.fori_loop(0, _NCHUNK, chunk, 0)
    plsc.subcore_barrier()

    base = c * _N + s * _ROWS_PER_TILE
    pltpu.sync_copy(acc.at[pl.ds(s * _ROWS_PER_TILE, _ROWS_PER_TILE)],
                    out_hbm.at[pl.ds(base, _ROWS_PER_TILE)])


def _sc_scatter(src3, dst3, table, zeros_hbm):
    mesh = plsc.VectorSubcoreMesh(core_axis_name="c", subcore_axis_name="s")
    f = pl.kernel(
        _sc_scatter_body,
        out_type=jax.ShapeDtypeStruct((2 * _N, _D), jnp.float32),
        mesh=mesh,
        scratch_types=[
            pltpu.VMEM((_NCHUNK, _CHUNK), jnp.int32),
            pltpu.VMEM((_NCHUNK, _CHUNK), jnp.int32),
            pltpu.VMEM((_CHUNK, _D), jnp.float32),
            pltpu.VMEM((_ROWS_PER_TILE, _D), jnp.float32),
            pltpu.VMEM_SHARED((_N, _D), jnp.float32),
            pltpu.SemaphoreType.DMA,
        ],
        compiler_params=pltpu.CompilerParams(use_tc_tiling_on_sc=False),
    )
    return f(src3, dst3, table, zeros_hbm)


def _prep_body(degp_ref, Y_ref, dv_ref, table1_ref):
    deg = degp_ref[0:_N, 0:1] + degp_ref[_N:2 * _N, 0:1] + 1.0
    dinv = lax.rsqrt(deg)
    table1_ref[...] = jnp.concatenate(
        [dinv * dv_ref[...], dinv * Y_ref[:, 0:3], dinv,
         jnp.zeros((_N, 11), jnp.float32)], axis=1)


def _mid_body(praw_ref, table1_ref, W1_ref, b1_ref, W2_ref, Wf_ref,
              table2_ref, s_ref):
    dinv = table1_ref[:, 4:5]
    p_full = dinv * (praw_ref[0:_N, :] + praw_ref[_N:2 * _N, :]) \
        + dinv * table1_ref[...]
    p = p_full[:, 0:4]
    s = p_full[:, 4:5]
    z = jnp.dot(p, W1_ref[...].T, preferred_element_type=jnp.float32,
                precision=lax.Precision.HIGHEST) + s * b1_ref[...]
    h = jnp.maximum(z, 0.0)
    C = jnp.dot(Wf_ref[...], W2_ref[...], preferred_element_type=jnp.float32,
                precision=lax.Precision.HIGHEST)
    g = jnp.dot(h, C.T, preferred_element_type=jnp.float32,
                precision=lax.Precision.HIGHEST)
    table2_ref[...] = jnp.concatenate(
        [dinv * g, jnp.zeros((_N, 14), jnp.float32)], axis=1)
    s_ref[...] = s


def _final_body(qraw_ref, table1_ref, table2_ref, s_ref, Wf_ref, b2_ref,
                bf_ref, out_ref):
    dinv = table1_ref[:, 4:5]
    qsum = qraw_ref[0:_N, 0:2] + qraw_ref[_N:2 * _N, 0:2]
    sb = jnp.dot(b2_ref[...], Wf_ref[...].T,
                 preferred_element_type=jnp.float32)
    out_ref[...] = dinv * qsum + dinv * table2_ref[:, 0:2] \
        + s_ref[...] * sb + bf_ref[...]


def _tc_single(body, out_shapes, *args):
    return pl.pallas_call(
        body,
        out_shape=out_shapes,
    )(*args)


def kernel(alpha, laplacian, num_node, threshold, diff_vec, edge_index,
           W1, b1, W2, b2, Wf, bf):
    n = diff_vec.shape[0]
    v = diff_vec.astype(jnp.float32)
    V3 = jnp.where(v < threshold, threshold, v)
    V4 = jnp.where(v >= threshold, threshold, v)
    V = jnp.concatenate(
        [v[:, None], V3[:, None], V4[:, None], jnp.zeros((n, 5), jnp.float32)],
        axis=1)

    Y = _neumann_solve(alpha, laplacian, V)

    src3 = edge_index[0].reshape(_NW, _NCHUNK, _CHUNK)
    dst3 = edge_index[1].reshape(_NW, _NCHUNK, _CHUNK)
    zeros_hbm = jnp.zeros((_ROWS_PER_TILE, _D), jnp.float32)
    ones_table = jnp.ones((_N, _D), jnp.float32)

    degp = _sc_scatter(src3, src3, ones_table, zeros_hbm)

    table1 = _tc_single(
        _prep_body, jax.ShapeDtypeStruct((_N, _D), jnp.float32),
        degp, Y, v[:, None])

    praw = _sc_scatter(src3, dst3, table1, zeros_hbm)

    table2, s = _tc_single(
        _mid_body,
        [jax.ShapeDtypeStruct((_N, _D), jnp.float32),
         jax.ShapeDtypeStruct((_N, 1), jnp.float32)],
        praw, table1, W1, jnp.reshape(b1, (1, 128)), W2, Wf)

    qraw = _sc_scatter(src3, dst3, table2, zeros_hbm)

    out = _tc_single(
        _final_body, jax.ShapeDtypeStruct((_N, 2), jnp.float32),
        qraw, table1, table2, s, Wf, jnp.reshape(b2, (1, 128)),
        jnp.reshape(bf, (1, 2)))

    return out + (jnp.asarray(num_node) - n).astype(out.dtype)


# speedup vs baseline: 3.5424x; 1.0158x over previous
import functools

import jax
import jax.numpy as jnp
from jax import lax
from jax.experimental import pallas as pl
from jax.experimental.pallas import tpu as pltpu
from jax.experimental.pallas import tpu_sc as plsc

_N = 4096
_E = 65536
_D = 16
_K_ITERS = 16
_BM = 512
_NW = 32
_CHUNK = 128
_NCHUNK = _E // (_NW * _CHUNK)
_ROWS_PER_TILE = _N // 16


_K_RES = 6
_K_REF = 2


def _cast_sweep_body(alpha_ref, L_ref, V_ref, Lhi_ref, Y1_ref):
    alpha = alpha_ref[0, 0]
    Lf = L_ref[...]
    Lhi_ref[...] = Lf.astype(jnp.bfloat16)
    Y1_ref[...] = V_ref[pl.ds(pl.program_id(0) * _BM, _BM), :] + alpha * (
        jnp.dot(Lf, V_ref[...], preferred_element_type=jnp.float32))


def _refine_body(alpha_ref, L_ref, V_ref, Yin_ref, out_ref, y0, y1):
    k = pl.program_id(0)
    i = pl.program_id(1)
    alpha = alpha_ref[0, 0]

    @pl.when(jnp.logical_and(k == 0, i == 0))
    def _():
        y0[...] = Yin_ref[...]

    def step(src, dst):
        Lf = L_ref[...]
        Lhi = Lf.astype(jnp.bfloat16)
        Llo = (Lf - Lhi.astype(jnp.float32)).astype(jnp.bfloat16)
        Ys = src[...]
        Yhi = Ys.astype(jnp.bfloat16)
        Ylo = (Ys - Yhi.astype(jnp.float32)).astype(jnp.bfloat16)
        acc = jnp.dot(Lhi, Yhi, preferred_element_type=jnp.float32)
        acc += jnp.dot(Lhi, Ylo, preferred_element_type=jnp.float32)
        acc += jnp.dot(Llo, Yhi, preferred_element_type=jnp.float32)
        new = V_ref[pl.ds(i * _BM, _BM), :] + alpha * acc
        dst[pl.ds(i * _BM, _BM), :] = new
        out_ref[...] = (1.0 - alpha) * new

    @pl.when(k % 2 == 0)
    def _():
        step(y0, y1)

    @pl.when(k % 2 == 1)
    def _():
        step(y1, y0)


def _neumann_solve(alpha, laplacian, V):
    alpha_s = jnp.reshape(alpha, (1, 1))
    Lhi, Y1 = pl.pallas_call(
        _cast_sweep_body,
        grid=(_N // _BM,),
        in_specs=[
            pl.BlockSpec(memory_space=pltpu.SMEM),
            pl.BlockSpec((_BM, _N), lambda i: (i, 0)),
            pl.BlockSpec((_N, 8), lambda i: (0, 0)),
        ],
        out_specs=[
            pl.BlockSpec((_BM, _N), lambda i: (i, 0)),
            pl.BlockSpec((_BM, 8), lambda i: (i, 0)),
        ],
        out_shape=[
            jax.ShapeDtypeStruct((_N, _N), jnp.bfloat16),
            jax.ShapeDtypeStruct((_N, 8), jnp.float32),
        ],
    )(alpha_s, laplacian, V)

    Y2 = pl.pallas_call(
        _resident_step_wrapper,
        grid=(_K_RES, _N // _BM),
        in_specs=[
            pl.BlockSpec(memory_space=pltpu.SMEM),
            pl.BlockSpec((_N, _N), lambda k, i: (0, 0)),
            pl.BlockSpec((_N, 8), lambda k, i: (0, 0)),
            pl.BlockSpec((_N, 8), lambda k, i: (0, 0)),
        ],
        out_specs=pl.BlockSpec((_BM, 8), lambda k, i: (i, 0)),
        out_shape=jax.ShapeDtypeStruct((_N, 8), jnp.float32),
        scratch_shapes=[
            pltpu.VMEM((_N, 8), jnp.float32),
            pltpu.VMEM((_N, 8), jnp.float32),
        ],
    )(alpha_s, Lhi, V, Y1)

    return pl.pallas_call(
        _refine_body,
        grid=(_K_REF, _N // _BM),
        in_specs=[
            pl.BlockSpec(memory_space=pltpu.SMEM),
            pl.BlockSpec((_BM, _N), lambda k, i: (i, 0)),
            pl.BlockSpec((_N, 8), lambda k, i: (0, 0)),
            pl.BlockSpec((_N, 8), lambda k, i: (0, 0)),
        ],
        out_specs=pl.BlockSpec((_BM, 8), lambda k, i: (i, 0)),
        out_shape=jax.ShapeDtypeStruct((_N, 8), jnp.float32),
        scratch_shapes=[
            pltpu.VMEM((_N, 8), jnp.float32),
            pltpu.VMEM((_N, 8), jnp.float32),
        ],
    )(alpha_s, laplacian, V, Y2)


def _resident_step_wrapper(alpha_ref, Lhi_ref, V_ref, Y1_ref, out_ref, y0, y1):
    k = pl.program_id(0)
    i = pl.program_id(1)
    alpha = alpha_ref[0, 0]

    @pl.when(jnp.logical_and(k == 0, i == 0))
    def _():
        y0[...] = Y1_ref[...]

    def step(src, dst):
        Lblk = Lhi_ref[pl.ds(i * _BM, _BM), :]
        acc = jnp.dot(Lblk, src[...].astype(jnp.bfloat16),
                      preferred_element_type=jnp.float32)
        new = V_ref[pl.ds(i * _BM, _BM), :] + alpha * acc
        dst[pl.ds(i * _BM, _BM), :] = new
        out_ref[...] = new

    @pl.when(k % 2 == 0)
    def _():
        step(y0, y1)

    @pl.when(k % 2 == 1)
    def _():
        step(y1, y0)


def _sc_scatter_body(gather, src_hbm, dst_hbm, table_hbm, zeros_hbm, out_hbm,
                     srcv, dstv, rows0, rows1, zrows, acc, sem0, sem1):
    c = lax.axis_index("c")
    s = lax.axis_index("s")
    wid = s * 2 + c

    pltpu.sync_copy(zeros_hbm, zrows)
    pltpu.sync_copy(zrows, acc.at[pl.ds(s * _ROWS_PER_TILE, _ROWS_PER_TILE)])
    plsc.subcore_barrier()

    pltpu.sync_copy(src_hbm.at[wid], srcv)
    pltpu.sync_copy(dst_hbm.at[wid], dstv)

    if gather:
        bufs = (rows0, rows1)
        sems = (sem0, sem1)
        pending = pltpu.async_copy(table_hbm.at[srcv.at[0]], rows0, sem0)
        for j in range(_NCHUNK):
            b = j % 2
            pending.wait()
            if j + 1 < _NCHUNK:
                pending = pltpu.async_copy(
                    table_hbm.at[srcv.at[j + 1]], bufs[1 - b], sems[1 - b])
            pltpu.sync_copy(bufs[b], acc.at[dstv.at[j]], add=True)
    else:
        pltpu.sync_copy(table_hbm, rows0)
        for j in range(_NCHUNK):
            pltpu.sync_copy(rows0, acc.at[dstv.at[j]], add=True)

    plsc.subcore_barrier()

    base = c * _N + s * _ROWS_PER_TILE
    pltpu.sync_copy(acc.at[pl.ds(s * _ROWS_PER_TILE, _ROWS_PER_TILE)],
                    out_hbm.at[pl.ds(base, _ROWS_PER_TILE)])


def _sc_scatter(src3, dst3, table, zeros_hbm, gather=True):
    mesh = plsc.VectorSubcoreMesh(core_axis_name="c", subcore_axis_name="s")
    f = pl.kernel(
        functools.partial(_sc_scatter_body, gather),
        out_type=jax.ShapeDtypeStruct((2 * _N, _D), jnp.float32),
        mesh=mesh,
        scratch_types=[
            pltpu.VMEM((_NCHUNK, _CHUNK), jnp.int32),
            pltpu.VMEM((_NCHUNK, _CHUNK), jnp.int32),
            pltpu.VMEM((_CHUNK, _D), jnp.float32),
            pltpu.VMEM((_CHUNK, _D), jnp.float32),
            pltpu.VMEM((_ROWS_PER_TILE, _D), jnp.float32),
            pltpu.VMEM_SHARED((_N, _D), jnp.float32),
            pltpu.SemaphoreType.DMA,
            pltpu.SemaphoreType.DMA,
        ],
        compiler_params=pltpu.CompilerParams(use_tc_tiling_on_sc=False),
    )
    return f(src3, dst3, table, zeros_hbm)


def _prep_body(degp_ref, Y_ref, dv_ref, table1_ref):
    deg = degp_ref[0:_N, 0:1] + degp_ref[_N:2 * _N, 0:1] + 1.0
    dinv = 1.0 / jnp.sqrt(deg)
    table1_ref[...] = jnp.concatenate(
        [dinv * dv_ref[...], dinv * Y_ref[:, 0:3], dinv,
         jnp.zeros((_N, 11), jnp.float32)], axis=1)


def _mid_body(praw_ref, table1_ref, W1_ref, b1_ref, W2_ref, Wf_ref,
              table2_ref, s_ref):
    dinv = table1_ref[:, 4:5]
    p_full = dinv * (praw_ref[0:_N, :] + praw_ref[_N:2 * _N, :]) \
        + dinv * table1_ref[...]
    p = p_full[:, 0:4]
    s = p_full[:, 4:5]
    z = jnp.dot(p, W1_ref[...].T, preferred_element_type=jnp.float32,
                precision=lax.Precision.HIGHEST) + s * b1_ref[...]
    h = jnp.maximum(z, 0.0)
    C = jnp.dot(Wf_ref[...], W2_ref[...], preferred_element_type=jnp.float32,
                precision=lax.Precision.HIGHEST)
    g = jnp.dot(h, C.T, preferred_element_type=jnp.float32,
                precision=lax.Precision.HIGHEST)
    table2_ref[...] = jnp.concatenate(
        [dinv * g, jnp.zeros((_N, 14), jnp.float32)], axis=1)
    s_ref[...] = s


def _final_body(qraw_ref, table1_ref, table2_ref, s_ref, Wf_ref, b2_ref,
                bf_ref, out_ref):
    dinv = table1_ref[:, 4:5]
    qsum = qraw_ref[0:_N, 0:2] + qraw_ref[_N:2 * _N, 0:2]
    sb = jnp.dot(b2_ref[...], Wf_ref[...].T,
                 preferred_element_type=jnp.float32)
    out_ref[...] = dinv * qsum + dinv * table2_ref[:, 0:2] \
        + s_ref[...] * sb + bf_ref[...]


def _tc_single(body, out_shapes, *args):
    return pl.pallas_call(
        body,
        out_shape=out_shapes,
    )(*args)


def kernel(alpha, laplacian, num_node, threshold, diff_vec, edge_index,
           W1, b1, W2, b2, Wf, bf):
    n = diff_vec.shape[0]
    v = diff_vec.astype(jnp.float32)
    V3 = jnp.where(v < threshold, threshold, v)
    V4 = jnp.where(v >= threshold, threshold, v)
    V = jnp.concatenate(
        [v[:, None], V3[:, None], V4[:, None], jnp.zeros((n, 5), jnp.float32)],
        axis=1)

    Y = _neumann_solve(alpha, laplacian, V)

    src3 = edge_index[0].reshape(_NW, _NCHUNK, _CHUNK)
    dst3 = edge_index[1].reshape(_NW, _NCHUNK, _CHUNK)
    zeros_hbm = jnp.zeros((_ROWS_PER_TILE, _D), jnp.float32)
    ones_block = jnp.ones((_CHUNK, _D), jnp.float32)

    degp = _sc_scatter(src3, src3, ones_block, zeros_hbm, gather=False)

    table1 = _tc_single(
        _prep_body, jax.ShapeDtypeStruct((_N, _D), jnp.float32),
        degp, Y, v[:, None])

    praw = _sc_scatter(src3, dst3, table1, zeros_hbm)

    table2, s = _tc_single(
        _mid_body,
        [jax.ShapeDtypeStruct((_N, _D), jnp.float32),
         jax.ShapeDtypeStruct((_N, 1), jnp.float32)],
        praw, table1, W1, jnp.reshape(b1, (1, 128)), W2, Wf)

    qraw = _sc_scatter(src3, dst3, table2, zeros_hbm)

    out = _tc_single(
        _final_body, jax.ShapeDtypeStruct((_N, 2), jnp.float32),
        qraw, table1, table2, s, Wf, jnp.reshape(b2, (1, 128)),
        jnp.reshape(bf, (1, 2)))

    return out + (jnp.asarray(num_node) - n).astype(out.dtype)


# speedup vs baseline: 3.9276x; 1.1087x over previous
import functools

import jax
import jax.numpy as jnp
from jax import lax
from jax.experimental import pallas as pl
from jax.experimental.pallas import tpu as pltpu
from jax.experimental.pallas import tpu_sc as plsc

_N = 4096
_E = 65536
_D = 16
_K_ITERS = 16
_BM = 512
_NW = 32
_CHUNK = 128
_NCHUNK = _E // (_NW * _CHUNK)
_ROWS_PER_TILE = _N // 16


_K_RES = 6
_K_REF = 1
_K_TOT = 1 + _K_RES + _K_REF
_BMS = 256


def _solver_body(alpha_ref, L_ref, V_ref, out_ref, Lhi, y0, y1):
    k = pl.program_id(0)
    i = pl.program_id(1)
    alpha = alpha_ref[0, 0]

    @pl.when(jnp.logical_and(k == 0, i == 0))
    def _():
        y0[...] = V_ref[...]

    def step(src, dst):
        @pl.when(k == 0)
        def _():
            Lf = L_ref[...]
            Lhi[pl.ds(i * _BMS, _BMS), :] = Lf.astype(jnp.bfloat16)
            new = V_ref[pl.ds(i * _BMS, _BMS), :] + alpha * jnp.dot(
                Lf, src[...], preferred_element_type=jnp.float32)
            dst[pl.ds(i * _BMS, _BMS), :] = new

        @pl.when(jnp.logical_and(k > 0, k <= _K_RES))
        def _():
            acc = jnp.dot(Lhi[pl.ds(i * _BMS, _BMS), :],
                          src[...].astype(jnp.bfloat16),
                          preferred_element_type=jnp.float32)
            new = V_ref[pl.ds(i * _BMS, _BMS), :] + alpha * acc
            dst[pl.ds(i * _BMS, _BMS), :] = new

        @pl.when(k > _K_RES)
        def _():
            Lf = L_ref[...]
            Lhib = Lhi[pl.ds(i * _BMS, _BMS), :]
            Llo = (Lf - Lhib.astype(jnp.float32)).astype(jnp.bfloat16)
            Ys = src[...]
            Yhi = Ys.astype(jnp.bfloat16)
            Ylo = (Ys - Yhi.astype(jnp.float32)).astype(jnp.bfloat16)
            acc = jnp.dot(Lhib, Yhi, preferred_element_type=jnp.float32)
            acc += jnp.dot(Lhib, Ylo, preferred_element_type=jnp.float32)
            acc += jnp.dot(Llo, Yhi, preferred_element_type=jnp.float32)
            new = V_ref[pl.ds(i * _BMS, _BMS), :] + alpha * acc
            dst[pl.ds(i * _BMS, _BMS), :] = new
            out_ref[...] = (1.0 - alpha) * new

    @pl.when(k % 2 == 0)
    def _():
        step(y0, y1)

    @pl.when(k % 2 == 1)
    def _():
        step(y1, y0)


def _neumann_solve(alpha, laplacian, V):
    def l_index(k, i):
        j = jnp.where(jnp.logical_or(k == 0, k > _K_RES), i, 0)
        return (j, 0)

    return pl.pallas_call(
        _solver_body,
        grid=(_K_TOT, _N // _BMS),
        in_specs=[
            pl.BlockSpec(memory_space=pltpu.SMEM),
            pl.BlockSpec((_BMS, _N), l_index),
            pl.BlockSpec((_N, 8), lambda k, i: (0, 0)),
        ],
        out_specs=pl.BlockSpec((_BMS, 8), lambda k, i: (i, 0)),
        out_shape=jax.ShapeDtypeStruct((_N, 8), jnp.float32),
        scratch_shapes=[
            pltpu.VMEM((_N, _N), jnp.bfloat16),
            pltpu.VMEM((_N, 8), jnp.float32),
            pltpu.VMEM((_N, 8), jnp.float32),
        ],
    )(jnp.reshape(alpha, (1, 1)), laplacian, V)


def _sc_scatter_body(gather, src_hbm, dst_hbm, table_hbm, zeros_hbm, out_hbm,
                     srcv, dstv, rows0, rows1, zrows, acc, sem0, sem1):
    c = lax.axis_index("c")
    s = lax.axis_index("s")
    wid = s * 2 + c

    pltpu.sync_copy(zeros_hbm, zrows)
    pltpu.sync_copy(zrows, acc.at[pl.ds(s * _ROWS_PER_TILE, _ROWS_PER_TILE)])
    plsc.subcore_barrier()

    pltpu.sync_copy(src_hbm.at[wid], srcv)
    pltpu.sync_copy(dst_hbm.at[wid], dstv)

    if gather:
        bufs = (rows0, rows1)
        sems = (sem0, sem1)
        pending = pltpu.async_copy(table_hbm.at[srcv.at[0]], rows0, sem0)
        for j in range(_NCHUNK):
            b = j % 2
            pending.wait()
            if j + 1 < _NCHUNK:
                pending = pltpu.async_copy(
                    table_hbm.at[srcv.at[j + 1]], bufs[1 - b], sems[1 - b])
            pltpu.sync_copy(bufs[b], acc.at[dstv.at[j]], add=True)
    else:
        pltpu.sync_copy(table_hbm, rows0)
        for j in range(_NCHUNK):
            pltpu.sync_copy(rows0, acc.at[dstv.at[j]], add=True)

    plsc.subcore_barrier()

    base = c * _N + s * _ROWS_PER_TILE
    pltpu.sync_copy(acc.at[pl.ds(s * _ROWS_PER_TILE, _ROWS_PER_TILE)],
                    out_hbm.at[pl.ds(base, _ROWS_PER_TILE)])


def _sc_scatter(src3, dst3, table, zeros_hbm, gather=True):
    mesh = plsc.VectorSubcoreMesh(core_axis_name="c", subcore_axis_name="s")
    f = pl.kernel(
        functools.partial(_sc_scatter_body, gather),
        out_type=jax.ShapeDtypeStruct((2 * _N, _D), jnp.float32),
        mesh=mesh,
        scratch_types=[
            pltpu.VMEM((_NCHUNK, _CHUNK), jnp.int32),
            pltpu.VMEM((_NCHUNK, _CHUNK), jnp.int32),
            pltpu.VMEM((_CHUNK, _D), jnp.float32),
            pltpu.VMEM((_CHUNK, _D), jnp.float32),
            pltpu.VMEM((_ROWS_PER_TILE, _D), jnp.float32),
            pltpu.VMEM_SHARED((_N, _D), jnp.float32),
            pltpu.SemaphoreType.DMA,
            pltpu.SemaphoreType.DMA,
        ],
        compiler_params=pltpu.CompilerParams(use_tc_tiling_on_sc=False),
    )
    return f(src3, dst3, table, zeros_hbm)


def _prep_body(degp_ref, Y_ref, dv_ref, table1_ref):
    deg = degp_ref[0:_N, 0:1] + degp_ref[_N:2 * _N, 0:1] + 1.0
    r = lax.rsqrt(deg)
    r = 0.5 * r * (3.0 - deg * r * r)
    dinv = 0.5 * r * (3.0 - deg * r * r)
    table1_ref[...] = jnp.concatenate(
        [dinv * dv_ref[...], dinv * Y_ref[:, 0:3], dinv,
         jnp.zeros((_N, 11), jnp.float32)], axis=1)


def _mid_body(praw_ref, table1_ref, W1_ref, b1_ref, W2_ref, Wf_ref,
              table2_ref, s_ref):
    dinv = table1_ref[:, 4:5]
    p_full = dinv * (praw_ref[0:_N, :] + praw_ref[_N:2 * _N, :]) \
        + dinv * table1_ref[...]
    p = p_full[:, 0:4]
    s = p_full[:, 4:5]
    z = jnp.dot(p, W1_ref[...].T, preferred_element_type=jnp.float32,
                precision=lax.Precision.HIGHEST) + s * b1_ref[...]
    h = jnp.maximum(z, 0.0)
    C = jnp.dot(Wf_ref[...], W2_ref[...], preferred_element_type=jnp.float32,
                precision=lax.Precision.HIGHEST)
    g = jnp.dot(h, C.T, preferred_element_type=jnp.float32,
                precision=lax.Precision.HIGHEST)
    table2_ref[...] = jnp.concatenate(
        [dinv * g, jnp.zeros((_N, 14), jnp.float32)], axis=1)
    s_ref[...] = s


def _final_body(qraw_ref, table1_ref, table2_ref, s_ref, Wf_ref, b2_ref,
                bf_ref, out_ref):
    dinv = table1_ref[:, 4:5]
    qsum = qraw_ref[0:_N, 0:2] + qraw_ref[_N:2 * _N, 0:2]
    sb = jnp.dot(b2_ref[...], Wf_ref[...].T,
                 preferred_element_type=jnp.float32)
    out_ref[...] = dinv * qsum + dinv * table2_ref[:, 0:2] \
        + s_ref[...] * sb + bf_ref[...]


def _tc_single(body, out_shapes, *args):
    return pl.pallas_call(
        body,
        out_shape=out_shapes,
    )(*args)


def kernel(alpha, laplacian, num_node, threshold, diff_vec, edge_index,
           W1, b1, W2, b2, Wf, bf):
    n = diff_vec.shape[0]
    v = diff_vec.astype(jnp.float32)
    V3 = jnp.where(v < threshold, threshold, v)
    V4 = jnp.where(v >= threshold, threshold, v)
    V = jnp.concatenate(
        [v[:, None], V3[:, None], V4[:, None], jnp.zeros((n, 5), jnp.float32)],
        axis=1)

    Y = _neumann_solve(alpha, laplacian, V)

    src3 = edge_index[0].reshape(_NW, _NCHUNK, _CHUNK)
    dst3 = edge_index[1].reshape(_NW, _NCHUNK, _CHUNK)
    zeros_hbm = jnp.zeros((_ROWS_PER_TILE, _D), jnp.float32)
    ones_block = jnp.ones((_CHUNK, _D), jnp.float32)

    degp = _sc_scatter(src3, src3, ones_block, zeros_hbm, gather=False)

    table1 = _tc_single(
        _prep_body, jax.ShapeDtypeStruct((_N, _D), jnp.float32),
        degp, Y, v[:, None])

    praw = _sc_scatter(src3, dst3, table1, zeros_hbm)

    table2, s = _tc_single(
        _mid_body,
        [jax.ShapeDtypeStruct((_N, _D), jnp.float32),
         jax.ShapeDtypeStruct((_N, 1), jnp.float32)],
        praw, table1, W1, jnp.reshape(b1, (1, 128)), W2, Wf)

    qraw = _sc_scatter(src3, dst3, table2, zeros_hbm)

    out = _tc_single(
        _final_body, jax.ShapeDtypeStruct((_N, 2), jnp.float32),
        qraw, table1, table2, s, Wf, jnp.reshape(b2, (1, 128)),
        jnp.reshape(bf, (1, 2)))

    return out + (jnp.asarray(num_node) - n).astype(out.dtype)


# speedup vs baseline: 4.1831x; 1.0651x over previous
import functools

import jax
import jax.numpy as jnp
from jax import lax
from jax.experimental import pallas as pl
from jax.experimental.pallas import tpu as pltpu
from jax.experimental.pallas import tpu_sc as plsc

_N = 4096
_E = 65536
_D = 16
_K_ITERS = 16
_BM = 512
_NW = 32
_CHUNK = 128
_NCHUNK = _E // (_NW * _CHUNK)
_ROWS_PER_TILE = _N // 16


_K_RES = 7
_K_REF = 0
_K_TOT = 1 + _K_RES + _K_REF
_BMS = 256


def _solver_body(alpha_ref, L_ref, V_ref, out_ref, Lhi, y0, y1):
    k = pl.program_id(0)
    i = pl.program_id(1)
    alpha = alpha_ref[0, 0]

    @pl.when(jnp.logical_and(k == 0, i == 0))
    def _():
        y0[...] = V_ref[...]

    def step(src, dst):
        @pl.when(k == 0)
        def _():
            Lf = L_ref[...]
            Lhi[pl.ds(i * _BMS, _BMS), :] = Lf.astype(jnp.bfloat16)
            new = V_ref[pl.ds(i * _BMS, _BMS), :] + alpha * jnp.dot(
                Lf, src[...], preferred_element_type=jnp.float32)
            dst[pl.ds(i * _BMS, _BMS), :] = new

        @pl.when(jnp.logical_and(k > 0, k <= _K_RES))
        def _():
            acc = jnp.dot(Lhi[pl.ds(i * _BMS, _BMS), :],
                          src[...].astype(jnp.bfloat16),
                          preferred_element_type=jnp.float32)
            new = V_ref[pl.ds(i * _BMS, _BMS), :] + alpha * acc
            dst[pl.ds(i * _BMS, _BMS), :] = new
            if _K_REF == 0:
                @pl.when(k == _K_TOT - 1)
                def _():
                    out_ref[...] = (1.0 - alpha) * new

        @pl.when(k > _K_RES)
        def _():
            Lf = L_ref[...]
            Lhib = Lhi[pl.ds(i * _BMS, _BMS), :]
            Llo = (Lf - Lhib.astype(jnp.float32)).astype(jnp.bfloat16)
            Ys = src[...]
            Yhi = Ys.astype(jnp.bfloat16)
            Ylo = (Ys - Yhi.astype(jnp.float32)).astype(jnp.bfloat16)
            acc = jnp.dot(Lhib, Yhi, preferred_element_type=jnp.float32)
            acc += jnp.dot(Lhib, Ylo, preferred_element_type=jnp.float32)
            acc += jnp.dot(Llo, Yhi, preferred_element_type=jnp.float32)
            new = V_ref[pl.ds(i * _BMS, _BMS), :] + alpha * acc
            dst[pl.ds(i * _BMS, _BMS), :] = new
            out_ref[...] = (1.0 - alpha) * new

    @pl.when(k % 2 == 0)
    def _():
        step(y0, y1)

    @pl.when(k % 2 == 1)
    def _():
        step(y1, y0)


def _neumann_solve(alpha, laplacian, V):
    def l_index(k, i):
        j = jnp.where(jnp.logical_or(k == 0, k > _K_RES), i, 0)
        return (j, 0)

    return pl.pallas_call(
        _solver_body,
        grid=(_K_TOT, _N // _BMS),
        in_specs=[
            pl.BlockSpec(memory_space=pltpu.SMEM),
            pl.BlockSpec((_BMS, _N), l_index),
            pl.BlockSpec((_N, 8), lambda k, i: (0, 0)),
        ],
        out_specs=pl.BlockSpec((_BMS, 8), lambda k, i: (i, 0)),
        out_shape=jax.ShapeDtypeStruct((_N, 8), jnp.float32),
        scratch_shapes=[
            pltpu.VMEM((_N, _N), jnp.bfloat16),
            pltpu.VMEM((_N, 8), jnp.float32),
            pltpu.VMEM((_N, 8), jnp.float32),
        ],
    )(jnp.reshape(alpha, (1, 1)), laplacian, V)


def _sc_scatter_body(gather, src_hbm, dst_hbm, table_hbm, zeros_hbm, out_hbm,
                     srcv, dstv, rows0, rows1, zrows, acc, sem0, sem1):
    c = lax.axis_index("c")
    s = lax.axis_index("s")
    wid = s * 2 + c

    pltpu.sync_copy(zeros_hbm, zrows)
    pltpu.sync_copy(zrows, acc.at[pl.ds(s * _ROWS_PER_TILE, _ROWS_PER_TILE)])
    plsc.subcore_barrier()

    pltpu.sync_copy(src_hbm.at[wid], srcv)
    pltpu.sync_copy(dst_hbm.at[wid], dstv)

    if gather:
        bufs = (rows0, rows1)
        sems = (sem0, sem1)
        pending = pltpu.async_copy(table_hbm.at[srcv.at[0]], rows0, sem0)
        for j in range(_NCHUNK):
            b = j % 2
            pending.wait()
            if j + 1 < _NCHUNK:
                pending = pltpu.async_copy(
                    table_hbm.at[srcv.at[j + 1]], bufs[1 - b], sems[1 - b])
            pltpu.sync_copy(bufs[b], acc.at[dstv.at[j]], add=True)
    else:
        pltpu.sync_copy(table_hbm, rows0)
        for j in range(_NCHUNK):
            pltpu.sync_copy(rows0, acc.at[dstv.at[j]], add=True)

    plsc.subcore_barrier()

    base = c * _N + s * _ROWS_PER_TILE
    pltpu.sync_copy(acc.at[pl.ds(s * _ROWS_PER_TILE, _ROWS_PER_TILE)],
                    out_hbm.at[pl.ds(base, _ROWS_PER_TILE)])


def _sc_scatter(src3, dst3, table, zeros_hbm, gather=True):
    mesh = plsc.VectorSubcoreMesh(core_axis_name="c", subcore_axis_name="s")
    f = pl.kernel(
        functools.partial(_sc_scatter_body, gather),
        out_type=jax.ShapeDtypeStruct((2 * _N, _D), jnp.float32),
        mesh=mesh,
        scratch_types=[
            pltpu.VMEM((_NCHUNK, _CHUNK), jnp.int32),
            pltpu.VMEM((_NCHUNK, _CHUNK), jnp.int32),
            pltpu.VMEM((_CHUNK, _D), jnp.float32),
            pltpu.VMEM((_CHUNK, _D), jnp.float32),
            pltpu.VMEM((_ROWS_PER_TILE, _D), jnp.float32),
            pltpu.VMEM_SHARED((_N, _D), jnp.float32),
            pltpu.SemaphoreType.DMA,
            pltpu.SemaphoreType.DMA,
        ],
        compiler_params=pltpu.CompilerParams(use_tc_tiling_on_sc=False),
    )
    return f(src3, dst3, table, zeros_hbm)


def _prep_body(degp_ref, Y_ref, dv_ref, table1_ref):
    deg = degp_ref[0:_N, 0:1] + degp_ref[_N:2 * _N, 0:1] + 1.0
    r = lax.rsqrt(deg)
    r = 0.5 * r * (3.0 - deg * r * r)
    dinv = 0.5 * r * (3.0 - deg * r * r)
    table1_ref[...] = jnp.concatenate(
        [dinv * dv_ref[...], dinv * Y_ref[:, 0:3], dinv,
         jnp.zeros((_N, 11), jnp.float32)], axis=1)


def _mid_body(praw_ref, table1_ref, W1_ref, b1_ref, W2_ref, Wf_ref,
              table2_ref, s_ref):
    dinv = table1_ref[:, 4:5]
    p_full = dinv * (praw_ref[0:_N, :] + praw_ref[_N:2 * _N, :]) \
        + dinv * table1_ref[...]
    p = p_full[:, 0:4]
    s = p_full[:, 4:5]
    z = jnp.dot(p, W1_ref[...].T, preferred_element_type=jnp.float32,
                precision=lax.Precision.HIGHEST) + s * b1_ref[...]
    h = jnp.maximum(z, 0.0)
    C = jnp.dot(Wf_ref[...], W2_ref[...], preferred_element_type=jnp.float32,
                precision=lax.Precision.HIGHEST)
    g = jnp.dot(h, C.T, preferred_element_type=jnp.float32,
                precision=lax.Precision.HIGHEST)
    table2_ref[...] = jnp.concatenate(
        [dinv * g, jnp.zeros((_N, 14), jnp.float32)], axis=1)
    s_ref[...] = s


def _final_body(qraw_ref, table1_ref, table2_ref, s_ref, Wf_ref, b2_ref,
                bf_ref, out_ref):
    dinv = table1_ref[:, 4:5]
    qsum = qraw_ref[0:_N, 0:2] + qraw_ref[_N:2 * _N, 0:2]
    sb = jnp.dot(b2_ref[...], Wf_ref[...].T,
                 preferred_element_type=jnp.float32)
    out_ref[...] = dinv * qsum + dinv * table2_ref[:, 0:2] \
        + s_ref[...] * sb + bf_ref[...]


def _tc_single(body, out_shapes, *args):
    return pl.pallas_call(
        body,
        out_shape=out_shapes,
    )(*args)


def kernel(alpha, laplacian, num_node, threshold, diff_vec, edge_index,
           W1, b1, W2, b2, Wf, bf):
    n = diff_vec.shape[0]
    v = diff_vec.astype(jnp.float32)
    V3 = jnp.where(v < threshold, threshold, v)
    V4 = jnp.where(v >= threshold, threshold, v)
    V = jnp.concatenate(
        [v[:, None], V3[:, None], V4[:, None], jnp.zeros((n, 5), jnp.float32)],
        axis=1)

    Y = _neumann_solve(alpha, laplacian, V)

    src3 = edge_index[0].reshape(_NW, _NCHUNK, _CHUNK)
    dst3 = edge_index[1].reshape(_NW, _NCHUNK, _CHUNK)
    zeros_hbm = jnp.zeros((_ROWS_PER_TILE, _D), jnp.float32)
    ones_block = jnp.ones((_CHUNK, _D), jnp.float32)

    degp = _sc_scatter(src3, src3, ones_block, zeros_hbm, gather=False)

    table1 = _tc_single(
        _prep_body, jax.ShapeDtypeStruct((_N, _D), jnp.float32),
        degp, Y, v[:, None])

    praw = _sc_scatter(src3, dst3, table1, zeros_hbm)

    table2, s = _tc_single(
        _mid_body,
        [jax.ShapeDtypeStruct((_N, _D), jnp.float32),
         jax.ShapeDtypeStruct((_N, 1), jnp.float32)],
        praw, table1, W1, jnp.reshape(b1, (1, 128)), W2, Wf)

    qraw = _sc_scatter(src3, dst3, table2, zeros_hbm)

    out = _tc_single(
        _final_body, jax.ShapeDtypeStruct((_N, 2), jnp.float32),
        qraw, table1, table2, s, Wf, jnp.reshape(b2, (1, 128)),
        jnp.reshape(bf, (1, 2)))

    return out + (jnp.asarray(num_node) - n).astype(out.dtype)


# speedup vs baseline: 4.4203x; 1.0567x over previous
import functools

import jax
import jax.numpy as jnp
from jax import lax
from jax.experimental import pallas as pl
from jax.experimental.pallas import tpu as pltpu
from jax.experimental.pallas import tpu_sc as plsc

_N = 4096
_E = 65536
_NW = 32
_CHUNK = 128
_NCHUNK = _E // (_NW * _CHUNK)
_ROWS_PER_TILE = _N // 16


_K_RES = 7
_K_REF = 0
_K_TOT = 1 + _K_RES + _K_REF
_BMS = 256


def _solver_body(alpha_ref, L_ref, V_ref, out_ref, Lhi, y0, y1):
    k = pl.program_id(0)
    i = pl.program_id(1)
    alpha = alpha_ref[0, 0]

    @pl.when(jnp.logical_and(k == 0, i == 0))
    def _():
        y0[...] = V_ref[...]

    def step(src, dst):
        @pl.when(k == 0)
        def _():
            Lf = L_ref[...]
            Lhi[pl.ds(i * _BMS, _BMS), :] = Lf.astype(jnp.bfloat16)
            new = V_ref[pl.ds(i * _BMS, _BMS), :] + alpha * jnp.dot(
                Lf, src[...], preferred_element_type=jnp.float32)
            dst[pl.ds(i * _BMS, _BMS), :] = new

        @pl.when(jnp.logical_and(k > 0, k <= _K_RES))
        def _():
            acc = jnp.dot(Lhi[pl.ds(i * _BMS, _BMS), :],
                          src[...].astype(jnp.bfloat16),
                          preferred_element_type=jnp.float32)
            new = V_ref[pl.ds(i * _BMS, _BMS), :] + alpha * acc
            dst[pl.ds(i * _BMS, _BMS), :] = new
            if _K_REF == 0:
                @pl.when(k == _K_TOT - 1)
                def _():
                    out_ref[...] = (1.0 - alpha) * new

        @pl.when(k > _K_RES)
        def _():
            Lf = L_ref[...]
            Lhib = Lhi[pl.ds(i * _BMS, _BMS), :]
            Llo = (Lf - Lhib.astype(jnp.float32)).astype(jnp.bfloat16)
            Ys = src[...]
            Yhi = Ys.astype(jnp.bfloat16)
            Ylo = (Ys - Yhi.astype(jnp.float32)).astype(jnp.bfloat16)
            acc = jnp.dot(Lhib, Yhi, preferred_element_type=jnp.float32)
            acc += jnp.dot(Lhib, Ylo, preferred_element_type=jnp.float32)
            acc += jnp.dot(Llo, Yhi, preferred_element_type=jnp.float32)
            new = V_ref[pl.ds(i * _BMS, _BMS), :] + alpha * acc
            dst[pl.ds(i * _BMS, _BMS), :] = new
            out_ref[...] = (1.0 - alpha) * new

    @pl.when(k % 2 == 0)
    def _():
        step(y0, y1)

    @pl.when(k % 2 == 1)
    def _():
        step(y1, y0)


def _neumann_solve(alpha, laplacian, V):
    def l_index(k, i):
        j = jnp.where(jnp.logical_or(k == 0, k > _K_RES), i, 0)
        return (j, 0)

    return pl.pallas_call(
        _solver_body,
        grid=(_K_TOT, _N // _BMS),
        in_specs=[
            pl.BlockSpec(memory_space=pltpu.SMEM),
            pl.BlockSpec((_BMS, _N), l_index),
            pl.BlockSpec((_N, 8), lambda k, i: (0, 0)),
        ],
        out_specs=pl.BlockSpec((_BMS, 8), lambda k, i: (i, 0)),
        out_shape=jax.ShapeDtypeStruct((_N, 8), jnp.float32),
        scratch_shapes=[
            pltpu.VMEM((_N, _N), jnp.bfloat16),
            pltpu.VMEM((_N, 8), jnp.float32),
            pltpu.VMEM((_N, 8), jnp.float32),
        ],
    )(jnp.reshape(alpha, (1, 1)), laplacian, V)


def _sc_scatter_body(gather, d, src_hbm, dst_hbm, table_hbm, zeros_hbm,
                     out_hbm, srcv, dstv, rows3, zrows, acc, gsem, ssem):
    c = lax.axis_index("c")
    s = lax.axis_index("s")
    wid = s * 2 + c

    pltpu.sync_copy(zeros_hbm, zrows)
    pltpu.sync_copy(zrows, acc.at[pl.ds(s * _ROWS_PER_TILE, _ROWS_PER_TILE)])
    plsc.subcore_barrier()

    pltpu.sync_copy(src_hbm.at[wid], srcv)
    pltpu.sync_copy(dst_hbm.at[wid], dstv)

    if gather:
        gcps = [pltpu.async_copy(table_hbm.at[srcv.at[j]], rows3.at[j], gsem)
                for j in range(_NCHUNK)]
        for cp in gcps:
            cp.wait()
        scps = [pltpu.async_copy(rows3.at[j], acc.at[dstv.at[j]], ssem,
                                 add=True)
                for j in range(_NCHUNK)]
        for cp in scps:
            cp.wait()
    else:
        pltpu.sync_copy(table_hbm, rows3.at[0])
        scps = [pltpu.async_copy(rows3.at[0], acc.at[dstv.at[j]], ssem,
                                 add=True)
                for j in range(_NCHUNK)]
        for cp in scps:
            cp.wait()

    plsc.subcore_barrier()

    base = c * _N + s * _ROWS_PER_TILE
    pltpu.sync_copy(acc.at[pl.ds(s * _ROWS_PER_TILE, _ROWS_PER_TILE)],
                    out_hbm.at[pl.ds(base, _ROWS_PER_TILE)])


def _sc_scatter(src3, dst3, table, zeros_hbm, gather=True):
    d = table.shape[1]
    mesh = plsc.VectorSubcoreMesh(core_axis_name="c", subcore_axis_name="s")
    f = pl.kernel(
        functools.partial(_sc_scatter_body, gather, d),
        out_type=jax.ShapeDtypeStruct((2 * _N, d), jnp.float32),
        mesh=mesh,
        scratch_types=[
            pltpu.VMEM((_NCHUNK, _CHUNK), jnp.int32),
            pltpu.VMEM((_NCHUNK, _CHUNK), jnp.int32),
            pltpu.VMEM((_NCHUNK, _CHUNK, d), jnp.float32),
            pltpu.VMEM((_ROWS_PER_TILE, d), jnp.float32),
            pltpu.VMEM_SHARED((_N, d), jnp.float32),
            pltpu.SemaphoreType.DMA,
            pltpu.SemaphoreType.DMA,
        ],
        compiler_params=pltpu.CompilerParams(use_tc_tiling_on_sc=False),
    )
    return f(src3, dst3, table, zeros_hbm)


def _prep_body(degp_ref, Y_ref, dv_ref, table1_ref):
    deg = degp_ref[0:_N, 0:1] + degp_ref[_N:2 * _N, 0:1] + 1.0
    r = lax.rsqrt(deg)
    r = 0.5 * r * (3.0 - deg * r * r)
    dinv = 0.5 * r * (3.0 - deg * r * r)
    table1_ref[...] = jnp.concatenate(
        [dinv * dv_ref[...], dinv * Y_ref[:, 0:3], dinv,
         jnp.zeros((_N, 3), jnp.float32)], axis=1)


def _mid_body(praw_ref, table1_ref, W1_ref, b1_ref, W2_ref, Wf_ref,
              table2_ref, s_ref):
    dinv = table1_ref[:, 4:5]
    p_full = dinv * (praw_ref[0:_N, :] + praw_ref[_N:2 * _N, :]) \
        + dinv * table1_ref[...]
    p = p_full[:, 0:4]
    s = p_full[:, 4:5]
    z = jnp.dot(p, W1_ref[...].T, preferred_element_type=jnp.float32,
                precision=lax.Precision.HIGHEST) + s * b1_ref[...]
    h = jnp.maximum(z, 0.0)
    C = jnp.dot(Wf_ref[...], W2_ref[...], preferred_element_type=jnp.float32,
                precision=lax.Precision.HIGHEST)
    g = jnp.dot(h, C.T, preferred_element_type=jnp.float32,
                precision=lax.Precision.HIGHEST)
    table2_ref[...] = jnp.concatenate(
        [dinv * g, jnp.zeros((_N, 2), jnp.float32)], axis=1)
    s_ref[...] = s


def _final_body(qraw_ref, table1_ref, table2_ref, s_ref, Wf_ref, b2_ref,
                bf_ref, out_ref):
    dinv = table1_ref[:, 4:5]
    qsum = qraw_ref[0:_N, 0:2] + qraw_ref[_N:2 * _N, 0:2]
    sb = jnp.dot(b2_ref[...], Wf_ref[...].T,
                 preferred_element_type=jnp.float32)
    out_ref[...] = dinv * qsum + dinv * table2_ref[:, 0:2] \
        + s_ref[...] * sb + bf_ref[...]


def _tc_single(body, out_shapes, *args):
    return pl.pallas_call(
        body,
        out_shape=out_shapes,
    )(*args)


def kernel(alpha, laplacian, num_node, threshold, diff_vec, edge_index,
           W1, b1, W2, b2, Wf, bf):
    n = diff_vec.shape[0]
    v = diff_vec.astype(jnp.float32)
    V3 = jnp.where(v < threshold, threshold, v)
    V4 = jnp.where(v >= threshold, threshold, v)
    V = jnp.concatenate(
        [v[:, None], V3[:, None], V4[:, None], jnp.zeros((n, 5), jnp.float32)],
        axis=1)

    Y = _neumann_solve(alpha, laplacian, V)

    src3 = edge_index[0].reshape(_NW, _NCHUNK, _CHUNK)
    dst3 = edge_index[1].reshape(_NW, _NCHUNK, _CHUNK)
    zeros8 = jnp.zeros((_ROWS_PER_TILE, 8), jnp.float32)
    zeros4 = jnp.zeros((_ROWS_PER_TILE, 4), jnp.float32)
    ones_block = jnp.ones((_CHUNK, 4), jnp.float32)

    degp = _sc_scatter(src3, src3, ones_block, zeros4, gather=False)

    table1 = _tc_single(
        _prep_body, jax.ShapeDtypeStruct((_N, 8), jnp.float32),
        degp, Y, v[:, None])

    praw = _sc_scatter(src3, dst3, table1, zeros8)

    table2, s = _tc_single(
        _mid_body,
        [jax.ShapeDtypeStruct((_N, 4), jnp.float32),
         jax.ShapeDtypeStruct((_N, 1), jnp.float32)],
        praw, table1, W1, jnp.reshape(b1, (1, 128)), W2, Wf)

    qraw = _sc_scatter(src3, dst3, table2, zeros4)

    out = _tc_single(
        _final_body, jax.ShapeDtypeStruct((_N, 2), jnp.float32),
        qraw, table1, table2, s, Wf, jnp.reshape(b2, (1, 128)),
        jnp.reshape(bf, (1, 2)))

    return out + (jnp.asarray(num_node) - n).astype(out.dtype)


# speedup vs baseline: 4.5596x; 1.0315x over previous
import functools

import jax
import jax.numpy as jnp
from jax import lax
from jax.experimental import pallas as pl
from jax.experimental.pallas import tpu as pltpu
from jax.experimental.pallas import tpu_sc as plsc

_N = 4096
_E = 65536
_NW = 32
_CHUNK = 128
_NCHUNK = _E // (_NW * _CHUNK)
_ROWS_PER_TILE = _N // 16


_K_RES = 7
_K_REF = 0
_K_TOT = 1 + _K_RES + _K_REF
_BMS = 256


def _solver_body(alpha_ref, L_ref, V_ref, out_ref, Lhi, y0, y1):
    k = pl.program_id(0)
    i = pl.program_id(1)
    alpha = alpha_ref[0, 0]

    @pl.when(jnp.logical_and(k == 0, i == 0))
    def _():
        y0[...] = V_ref[...]

    def step(src, dst):
        @pl.when(k == 0)
        def _():
            Lf = L_ref[...]
            Lhi[pl.ds(i * _BMS, _BMS), :] = Lf.astype(jnp.bfloat16)
            new = V_ref[pl.ds(i * _BMS, _BMS), :] + alpha * jnp.dot(
                Lf, src[...], preferred_element_type=jnp.float32)
            dst[pl.ds(i * _BMS, _BMS), :] = new

        @pl.when(jnp.logical_and(k > 0, k <= _K_RES))
        def _():
            acc = jnp.dot(Lhi[pl.ds(i * _BMS, _BMS), :],
                          src[...].astype(jnp.bfloat16),
                          preferred_element_type=jnp.float32)
            new = V_ref[pl.ds(i * _BMS, _BMS), :] + alpha * acc
            dst[pl.ds(i * _BMS, _BMS), :] = new
            if _K_REF == 0:
                @pl.when(k == _K_TOT - 1)
                def _():
                    out_ref[...] = (1.0 - alpha) * new

        @pl.when(k > _K_RES)
        def _():
            Lf = L_ref[...]
            Lhib = Lhi[pl.ds(i * _BMS, _BMS), :]
            Llo = (Lf - Lhib.astype(jnp.float32)).astype(jnp.bfloat16)
            Ys = src[...]
            Yhi = Ys.astype(jnp.bfloat16)
            Ylo = (Ys - Yhi.astype(jnp.float32)).astype(jnp.bfloat16)
            acc = jnp.dot(Lhib, Yhi, preferred_element_type=jnp.float32)
            acc += jnp.dot(Lhib, Ylo, preferred_element_type=jnp.float32)
            acc += jnp.dot(Llo, Yhi, preferred_element_type=jnp.float32)
            new = V_ref[pl.ds(i * _BMS, _BMS), :] + alpha * acc
            dst[pl.ds(i * _BMS, _BMS), :] = new
            out_ref[...] = (1.0 - alpha) * new

    @pl.when(k % 2 == 0)
    def _():
        step(y0, y1)

    @pl.when(k % 2 == 1)
    def _():
        step(y1, y0)


def _neumann_solve(alpha, laplacian, V):
    def l_index(k, i):
        j = jnp.where(jnp.logical_or(k == 0, k > _K_RES), i, 0)
        return (j, 0)

    return pl.pallas_call(
        _solver_body,
        grid=(_K_TOT, _N // _BMS),
        in_specs=[
            pl.BlockSpec(memory_space=pltpu.SMEM),
            pl.BlockSpec((_BMS, _N), l_index),
            pl.BlockSpec((_N, 8), lambda k, i: (0, 0)),
        ],
        out_specs=pl.BlockSpec((_BMS, 8), lambda k, i: (i, 0)),
        out_shape=jax.ShapeDtypeStruct((_N, 8), jnp.float32),
        scratch_shapes=[
            pltpu.VMEM((_N, _N), jnp.bfloat16),
            pltpu.VMEM((_N, 8), jnp.float32),
            pltpu.VMEM((_N, 8), jnp.float32),
        ],
    )(jnp.reshape(alpha, (1, 1)), laplacian, V)


def _sc_scatter_body(gather, d, src_hbm, dst_hbm, table_hbm, zeros_hbm,
                     out_hbm, srcv, dstv, rows3, zrows, acc, gsem, ssem):
    c = lax.axis_index("c")
    s = lax.axis_index("s")
    wid = s * 2 + c

    pltpu.sync_copy(zeros_hbm, zrows)
    pltpu.sync_copy(zrows, acc.at[pl.ds(s * _ROWS_PER_TILE, _ROWS_PER_TILE)])
    plsc.subcore_barrier()

    pltpu.sync_copy(src_hbm.at[wid], srcv)
    pltpu.sync_copy(dst_hbm.at[wid], dstv)

    if gather:
        gcps = [pltpu.async_copy(table_hbm.at[srcv.at[j]], rows3.at[j], gsem)
                for j in range(_NCHUNK)]
        for cp in gcps:
            cp.wait()
        scps = [pltpu.async_copy(rows3.at[j], acc.at[dstv.at[j]], ssem,
                                 add=True)
                for j in range(_NCHUNK)]
        for cp in scps:
            cp.wait()
    else:
        pltpu.sync_copy(table_hbm, rows3.at[0])
        scps = [pltpu.async_copy(rows3.at[0], acc.at[dstv.at[j]], ssem,
                                 add=True)
                for j in range(_NCHUNK)]
        for cp in scps:
            cp.wait()

    plsc.subcore_barrier()

    base = c * _N + s * _ROWS_PER_TILE
    pltpu.sync_copy(acc.at[pl.ds(s * _ROWS_PER_TILE, _ROWS_PER_TILE)],
                    out_hbm.at[pl.ds(base, _ROWS_PER_TILE)])


def _sc_scatter(src3, dst3, table, zeros_hbm, gather=True):
    d = table.shape[1]
    mesh = plsc.VectorSubcoreMesh(core_axis_name="c", subcore_axis_name="s")
    f = pl.kernel(
        functools.partial(_sc_scatter_body, gather, d),
        out_type=jax.ShapeDtypeStruct((2 * _N, d), jnp.float32),
        mesh=mesh,
        scratch_types=[
            pltpu.VMEM((_NCHUNK, _CHUNK), jnp.int32),
            pltpu.VMEM((_NCHUNK, _CHUNK), jnp.int32),
            pltpu.VMEM((_NCHUNK, _CHUNK, d), jnp.float32),
            pltpu.VMEM((_ROWS_PER_TILE, d), jnp.float32),
            pltpu.VMEM_SHARED((_N, d), jnp.float32),
            pltpu.SemaphoreType.DMA,
            pltpu.SemaphoreType.DMA,
        ],
        compiler_params=pltpu.CompilerParams(use_tc_tiling_on_sc=False),
    )
    return f(src3, dst3, table, zeros_hbm)


def _prep_body(degp_ref, Y_ref, dv_ref, table1_ref):
    deg = degp_ref[0:_N, 0:1] + degp_ref[_N:2 * _N, 0:1] + 1.0
    r = lax.rsqrt(deg)
    r = 0.5 * r * (3.0 - deg * r * r)
    dinv = 0.5 * r * (3.0 - deg * r * r)
    table1_ref[...] = jnp.concatenate(
        [dinv * dv_ref[...], dinv * Y_ref[:, 0:3], dinv,
         jnp.zeros((_N, 11), jnp.float32)], axis=1)


def _mid_body(praw_ref, table1_ref, W1_ref, b1_ref, W2_ref, Wf_ref,
              table2_ref, s_ref):
    dinv = table1_ref[:, 4:5]
    p_full = dinv * (praw_ref[0:_N, :] + praw_ref[_N:2 * _N, :]) \
        + dinv * table1_ref[...]
    p = p_full[:, 0:4]
    s = p_full[:, 4:5]
    z = jnp.dot(p, W1_ref[...].T, preferred_element_type=jnp.float32,
                precision=lax.Precision.HIGHEST) + s * b1_ref[...]
    h = jnp.maximum(z, 0.0)
    C = jnp.dot(Wf_ref[...], W2_ref[...], preferred_element_type=jnp.float32,
                precision=lax.Precision.HIGHEST)
    g = jnp.dot(h, C.T, preferred_element_type=jnp.float32,
                precision=lax.Precision.HIGHEST)
    table2_ref[...] = jnp.concatenate(
        [dinv * g, jnp.zeros((_N, 14), jnp.float32)], axis=1)
    s_ref[...] = s


def _final_body(qraw_ref, table1_ref, table2_ref, s_ref, Wf_ref, b2_ref,
                bf_ref, out_ref):
    dinv = table1_ref[:, 4:5]
    qsum = qraw_ref[0:_N, 0:2] + qraw_ref[_N:2 * _N, 0:2]
    sb = jnp.dot(b2_ref[...], Wf_ref[...].T,
                 preferred_element_type=jnp.float32)
    out_ref[...] = dinv * qsum + dinv * table2_ref[:, 0:2] \
        + s_ref[...] * sb + bf_ref[...]


def _tc_single(body, out_shapes, *args):
    return pl.pallas_call(
        body,
        out_shape=out_shapes,
    )(*args)


def kernel(alpha, laplacian, num_node, threshold, diff_vec, edge_index,
           W1, b1, W2, b2, Wf, bf):
    n = diff_vec.shape[0]
    v = diff_vec.astype(jnp.float32)
    V3 = jnp.where(v < threshold, threshold, v)
    V4 = jnp.where(v >= threshold, threshold, v)
    V = jnp.concatenate(
        [v[:, None], V3[:, None], V4[:, None], jnp.zeros((n, 5), jnp.float32)],
        axis=1)

    Y = _neumann_solve(alpha, laplacian, V)

    src3 = edge_index[0].reshape(_NW, _NCHUNK, _CHUNK)
    dst3 = edge_index[1].reshape(_NW, _NCHUNK, _CHUNK)
    zeros16 = jnp.zeros((_ROWS_PER_TILE, 16), jnp.float32)
    ones_block = jnp.ones((_CHUNK, 16), jnp.float32)

    degp = _sc_scatter(src3, src3, ones_block, zeros16, gather=False)

    table1 = _tc_single(
        _prep_body, jax.ShapeDtypeStruct((_N, 16), jnp.float32),
        degp, Y, v[:, None])

    praw = _sc_scatter(src3, dst3, table1, zeros16)

    table2, s = _tc_single(
        _mid_body,
        [jax.ShapeDtypeStruct((_N, 16), jnp.float32),
         jax.ShapeDtypeStruct((_N, 1), jnp.float32)],
        praw, table1, W1, jnp.reshape(b1, (1, 128)), W2, Wf)

    qraw = _sc_scatter(src3, dst3, table2, zeros16)

    out = _tc_single(
        _final_body, jax.ShapeDtypeStruct((_N, 2), jnp.float32),
        qraw, table1, table2, s, Wf, jnp.reshape(b2, (1, 128)),
        jnp.reshape(bf, (1, 2)))

    return out + (jnp.asarray(num_node) - n).astype(out.dtype)


# speedup vs baseline: 4.9207x; 1.0792x over previous
import functools

import jax
import jax.numpy as jnp
from jax import lax
from jax.experimental import pallas as pl
from jax.experimental.pallas import tpu as pltpu
from jax.experimental.pallas import tpu_sc as plsc

_N = 4096
_E = 65536
_NW = 32
_CHUNK = 128
_NCHUNK = _E // (_NW * _CHUNK)
_ROWS_PER_TILE = _N // 16


_K_RES = 6
_K_REF = 0
_K_TOT = 1 + _K_RES + _K_REF
_BMS = 256


def _solver_body(alpha_ref, thr_ref, L_ref, dv_ref, out_ref, Lhi, V_ref,
                 y0, y1):
    k = pl.program_id(0)
    i = pl.program_id(1)
    alpha = alpha_ref[0, 0]

    @pl.when(jnp.logical_and(k == 0, i == 0))
    def _():
        thr = thr_ref[0, 0]
        dv = dv_ref[...]
        V_ref[...] = jnp.concatenate(
            [dv, jnp.where(dv < thr, thr, dv), jnp.where(dv >= thr, thr, dv),
             jnp.zeros((_N, 5), jnp.float32)], axis=1)
        y0[...] = V_ref[...]

    def step(src, dst):
        @pl.when(k == 0)
        def _():
            Lf = L_ref[...]
            Lhi[pl.ds(i * _BMS, _BMS), :] = Lf.astype(jnp.bfloat16)
            new = V_ref[pl.ds(i * _BMS, _BMS), :] + alpha * jnp.dot(
                Lf, src[...], preferred_element_type=jnp.float32)
            dst[pl.ds(i * _BMS, _BMS), :] = new

        @pl.when(jnp.logical_and(k > 0, k <= _K_RES))
        def _():
            acc = jnp.dot(Lhi[pl.ds(i * _BMS, _BMS), :],
                          src[...].astype(jnp.bfloat16),
                          preferred_element_type=jnp.float32)
            new = V_ref[pl.ds(i * _BMS, _BMS), :] + alpha * acc
            dst[pl.ds(i * _BMS, _BMS), :] = new
            if _K_REF == 0:
                @pl.when(k == _K_TOT - 1)
                def _():
                    out_ref[...] = (1.0 - alpha) * new

        @pl.when(k > _K_RES)
        def _():
            Lf = L_ref[...]
            Lhib = Lhi[pl.ds(i * _BMS, _BMS), :]
            Llo = (Lf - Lhib.astype(jnp.float32)).astype(jnp.bfloat16)
            Ys = src[...]
            Yhi = Ys.astype(jnp.bfloat16)
            Ylo = (Ys - Yhi.astype(jnp.float32)).astype(jnp.bfloat16)
            acc = jnp.dot(Lhib, Yhi, preferred_element_type=jnp.float32)
            acc += jnp.dot(Lhib, Ylo, preferred_element_type=jnp.float32)
            acc += jnp.dot(Llo, Yhi, preferred_element_type=jnp.float32)
            new = V_ref[pl.ds(i * _BMS, _BMS), :] + alpha * acc
            dst[pl.ds(i * _BMS, _BMS), :] = new
            out_ref[...] = (1.0 - alpha) * new

    @pl.when(k % 2 == 0)
    def _():
        step(y0, y1)

    @pl.when(k % 2 == 1)
    def _():
        step(y1, y0)


def _neumann_solve(alpha, threshold, laplacian, diff_vec):
    def l_index(k, i):
        j = jnp.where(jnp.logical_or(k == 0, k > _K_RES), i, 0)
        return (j, 0)

    return pl.pallas_call(
        _solver_body,
        grid=(_K_TOT, _N // _BMS),
        in_specs=[
            pl.BlockSpec(memory_space=pltpu.SMEM),
            pl.BlockSpec(memory_space=pltpu.SMEM),
            pl.BlockSpec((_BMS, _N), l_index),
            pl.BlockSpec((_N, 1), lambda k, i: (0, 0)),
        ],
        out_specs=pl.BlockSpec((_BMS, 8), lambda k, i: (i, 0)),
        out_shape=jax.ShapeDtypeStruct((_N, 8), jnp.float32),
        scratch_shapes=[
            pltpu.VMEM((_N, _N), jnp.bfloat16),
            pltpu.VMEM((_N, 8), jnp.float32),
            pltpu.VMEM((_N, 8), jnp.float32),
            pltpu.VMEM((_N, 8), jnp.float32),
        ],
    )(jnp.reshape(alpha, (1, 1)), jnp.reshape(threshold, (1, 1)),
      laplacian, diff_vec[:, None])


def _sc_scatter_body(gather, d, src_hbm, dst_hbm, table_hbm, zeros_hbm,
                     out_hbm, srcv, dstv, rows3, zrows, acc, gsem, ssem):
    c = lax.axis_index("c")
    s = lax.axis_index("s")
    wid = s * 2 + c

    pltpu.sync_copy(zeros_hbm, zrows)
    pltpu.sync_copy(zrows, acc.at[pl.ds(s * _ROWS_PER_TILE, _ROWS_PER_TILE)])
    plsc.subcore_barrier()

    pltpu.sync_copy(src_hbm.at[wid], srcv)
    pltpu.sync_copy(dst_hbm.at[wid], dstv)

    if gather:
        gcps = [pltpu.async_copy(table_hbm.at[srcv.at[j]], rows3.at[j], gsem)
                for j in range(_NCHUNK)]
        for cp in gcps:
            cp.wait()
        scps = [pltpu.async_copy(rows3.at[j], acc.at[dstv.at[j]], ssem,
                                 add=True)
                for j in range(_NCHUNK)]
        for cp in scps:
            cp.wait()
    else:
        pltpu.sync_copy(table_hbm, rows3.at[0])
        scps = [pltpu.async_copy(rows3.at[0], acc.at[dstv.at[j]], ssem,
                                 add=True)
                for j in range(_NCHUNK)]
        for cp in scps:
            cp.wait()

    plsc.subcore_barrier()

    base = c * _N + s * _ROWS_PER_TILE
    pltpu.sync_copy(acc.at[pl.ds(s * _ROWS_PER_TILE, _ROWS_PER_TILE)],
                    out_hbm.at[pl.ds(base, _ROWS_PER_TILE)])


def _sc_scatter(src3, dst3, table, zeros_hbm, gather=True):
    d = table.shape[1]
    mesh = plsc.VectorSubcoreMesh(core_axis_name="c", subcore_axis_name="s")
    f = pl.kernel(
        functools.partial(_sc_scatter_body, gather, d),
        out_type=jax.ShapeDtypeStruct((2 * _N, d), jnp.float32),
        mesh=mesh,
        scratch_types=[
            pltpu.VMEM((_NCHUNK, _CHUNK), jnp.int32),
            pltpu.VMEM((_NCHUNK, _CHUNK), jnp.int32),
            pltpu.VMEM((_NCHUNK, _CHUNK, d), jnp.float32),
            pltpu.VMEM((_ROWS_PER_TILE, d), jnp.float32),
            pltpu.VMEM_SHARED((_N, d), jnp.float32),
            pltpu.SemaphoreType.DMA,
            pltpu.SemaphoreType.DMA,
        ],
        compiler_params=pltpu.CompilerParams(use_tc_tiling_on_sc=False),
    )
    return f(src3, dst3, table, zeros_hbm)


def _prep_body(degp_ref, Y_ref, dv_ref, table1_ref):
    deg = degp_ref[0:_N, 0:1] + degp_ref[_N:2 * _N, 0:1] + 1.0
    r = lax.rsqrt(deg)
    r = 0.5 * r * (3.0 - deg * r * r)
    dinv = 0.5 * r * (3.0 - deg * r * r)
    table1_ref[...] = jnp.concatenate(
        [dinv * dv_ref[...], dinv * Y_ref[:, 0:3], dinv,
         jnp.zeros((_N, 11), jnp.float32)], axis=1)


def _mid_body(praw_ref, table1_ref, W1_ref, b1_ref, W2_ref, Wf_ref,
              table2_ref, s_ref):
    dinv = table1_ref[:, 4:5]
    p_full = dinv * (praw_ref[0:_N, :] + praw_ref[_N:2 * _N, :]) \
        + dinv * table1_ref[...]
    p = p_full[:, 0:4]
    s = p_full[:, 4:5]
    z = jnp.dot(p, W1_ref[...].T, preferred_element_type=jnp.float32,
                precision=lax.Precision.HIGHEST) + s * b1_ref[...]
    h = jnp.maximum(z, 0.0)
    C = jnp.dot(Wf_ref[...], W2_ref[...], preferred_element_type=jnp.float32,
                precision=lax.Precision.HIGHEST)
    g = jnp.dot(h, C.T, preferred_element_type=jnp.float32,
                precision=lax.Precision.HIGHEST)
    table2_ref[...] = jnp.concatenate(
        [dinv * g, jnp.zeros((_N, 14), jnp.float32)], axis=1)
    s_ref[...] = s


def _final_body(off_ref, qraw_ref, table1_ref, table2_ref, s_ref, Wf_ref,
                b2_ref, bf_ref, out_ref):
    dinv = table1_ref[:, 4:5]
    qsum = qraw_ref[0:_N, 0:2] + qraw_ref[_N:2 * _N, 0:2]
    sb = jnp.dot(b2_ref[...], Wf_ref[...].T,
                 preferred_element_type=jnp.float32)
    out_ref[...] = dinv * qsum + dinv * table2_ref[:, 0:2] \
        + s_ref[...] * sb + bf_ref[...] + off_ref[0, 0]


def _tc_single(body, out_shapes, *args):
    return pl.pallas_call(
        body,
        out_shape=out_shapes,
    )(*args)


def kernel(alpha, laplacian, num_node, threshold, diff_vec, edge_index,
           W1, b1, W2, b2, Wf, bf):
    n = diff_vec.shape[0]
    v = diff_vec.astype(jnp.float32)

    Y = _neumann_solve(alpha, threshold, laplacian, v)

    src3 = edge_index[0].reshape(_NW, _NCHUNK, _CHUNK)
    dst3 = edge_index[1].reshape(_NW, _NCHUNK, _CHUNK)
    zeros16 = jnp.zeros((_ROWS_PER_TILE, 16), jnp.float32)
    ones_block = jnp.ones((_CHUNK, 16), jnp.float32)

    degp = _sc_scatter(src3, src3, ones_block, zeros16, gather=False)

    table1 = _tc_single(
        _prep_body, jax.ShapeDtypeStruct((_N, 16), jnp.float32),
        degp, Y, v[:, None])

    praw = _sc_scatter(src3, dst3, table1, zeros16)

    table2, s = _tc_single(
        _mid_body,
        [jax.ShapeDtypeStruct((_N, 16), jnp.float32),
         jax.ShapeDtypeStruct((_N, 1), jnp.float32)],
        praw, table1, W1, jnp.reshape(b1, (1, 128)), W2, Wf)

    qraw = _sc_scatter(src3, dst3, table2, zeros16)

    off = (jnp.asarray(num_node) - n).astype(jnp.float32)
    return _tc_single(
        _final_body, jax.ShapeDtypeStruct((_N, 2), jnp.float32),
        jnp.reshape(off, (1, 1)), qraw, table1, table2, s, Wf,
        jnp.reshape(b2, (1, 128)), jnp.reshape(bf, (1, 2)))
